# Initial kernel scaffold; baseline (speedup 1.0000x reference)
import jax
import jax.numpy as jnp
from jax.experimental import pallas as pl


def kernel(time_features, month_table, day_table, weekday_table):
    raise NotImplementedError("write your pallas kernel here")


# speedup vs baseline: 1.4473x; 1.4473x over previous
import functools

import jax
import jax.numpy as jnp
from jax import lax
from jax.experimental import pallas as pl
from jax.experimental.pallas import tpu as pltpu
from jax.experimental.pallas import tpu_sc as plsc

NC = 2
NS = 16
NW = NC * NS
L = 16

D_MODEL = 128
MONTH_SIZE = 13
DAY_SIZE = 32
COMB = MONTH_SIZE * DAY_SIZE

BATCH = 4096
SEQ = 200
N_ROWS = BATCH * SEQ
ROWS_PER_TILE = N_ROWS // NW
GROUP = 128
NGROUPS = ROWS_PER_TILE // GROUP


def _sc_body(tf_hbm, month_hbm, day_hbm, out_hbm, comb_hbm,
             month_v, day_v, comb_v, tf_v, idx_v, rows_v, sem):
    cid = lax.axis_index("c")
    sid = lax.axis_index("s")
    wid = sid * NC + cid
    base = wid * ROWS_PER_TILE

    @pl.when(sid == 0)
    def _build():
        pltpu.sync_copy(month_hbm, month_v)
        pltpu.sync_copy(day_hbm, day_v)

        def mloop(m, carry):
            for ch in range(D_MODEL // L):
                sl = pl.ds(ch * L, L)
                mv = month_v[m, sl]
                for dd in range(DAY_SIZE):
                    comb_v[m * DAY_SIZE + dd, sl] = mv + day_v[dd, sl]
            return carry

        lax.fori_loop(0, MONTH_SIZE, mloop, 0)
        pltpu.sync_copy(comb_v, comb_hbm)

    plsc.subcore_barrier()

    lanes = lax.iota(jnp.int32, L)

    def gloop(j, carry):
        row0 = base + j * GROUP
        pltpu.sync_copy(tf_hbm.at[pl.ds(row0 * 3, GROUP * 3)], tf_v)
        for k in range(GROUP // L):
            pos = lanes * 3 + (k * L * 3)
            m = plsc.load_gather(tf_v, [pos])
            d = plsc.load_gather(tf_v, [pos + 1])
            idx_v[pl.ds(k * L, L)] = m * DAY_SIZE + d
        pltpu.async_copy(comb_hbm.at[idx_v], rows_v, sem).wait()
        pltpu.sync_copy(rows_v, out_hbm.at[pl.ds(row0, GROUP)])
        return carry

    lax.fori_loop(0, NGROUPS, gloop, 0)


@functools.partial(
    pl.kernel,
    out_type=[
        jax.ShapeDtypeStruct((N_ROWS, D_MODEL), jnp.float32),
        jax.ShapeDtypeStruct((COMB, D_MODEL), jnp.float32),
    ],
    mesh=plsc.VectorSubcoreMesh(core_axis_name="c", subcore_axis_name="s"),
    compiler_params=pltpu.CompilerParams(needs_layout_passes=False),
    scratch_types=[
        pltpu.VMEM((MONTH_SIZE, D_MODEL), jnp.float32),
        pltpu.VMEM((DAY_SIZE, D_MODEL), jnp.float32),
        pltpu.VMEM((COMB, D_MODEL), jnp.float32),
        pltpu.VMEM((GROUP * 3,), jnp.int32),
        pltpu.VMEM((GROUP,), jnp.int32),
        pltpu.VMEM((GROUP, D_MODEL), jnp.float32),
        pltpu.SemaphoreType.DMA,
    ],
)
def _sc_embed(tf_hbm, month_hbm, day_hbm, out_hbm, comb_hbm, *scratch):
    _sc_body(tf_hbm, month_hbm, day_hbm, out_hbm, comb_hbm, *scratch)


def kernel(time_features, month_table, day_table, weekday_table):
    tf = time_features.astype(jnp.int32).reshape(-1)
    out, _ = _sc_embed(tf, month_table, day_table)
    return out.reshape(BATCH, SEQ, D_MODEL)


# speedup vs baseline: 1.4506x; 1.0023x over previous
import functools

import jax
import jax.numpy as jnp
from jax import lax
from jax.experimental import pallas as pl
from jax.experimental.pallas import tpu as pltpu
from jax.experimental.pallas import tpu_sc as plsc

NC = 2
NS = 16
NW = NC * NS
L = 16

D_MODEL = 128
MONTH_SIZE = 13
DAY_SIZE = 32
COMB = MONTH_SIZE * DAY_SIZE

BATCH = 4096
SEQ = 200
N_ROWS = BATCH * SEQ
ROWS_PER_TILE = N_ROWS // NW
GROUP = 128
NGROUPS = ROWS_PER_TILE // GROUP
NBUF = 2


def _sc_body(tf_hbm, month_hbm, day_hbm, out_hbm, comb_hbm,
             month_v, day_v, comb_v,
             tf0, tf1, idx0, idx1, rows0, rows1,
             sem_tf0, sem_tf1, sem_g0, sem_g1, sem_w0, sem_w1):
    tf_v = (tf0, tf1)
    idx_v = (idx0, idx1)
    rows_v = (rows0, rows1)
    sem_tf = (sem_tf0, sem_tf1)
    sem_g = (sem_g0, sem_g1)
    sem_wr = (sem_w0, sem_w1)
    cid = lax.axis_index("c")
    sid = lax.axis_index("s")
    wid = sid * NC + cid
    base = wid * ROWS_PER_TILE

    @pl.when(sid == 0)
    def _build():
        pltpu.sync_copy(month_hbm, month_v)
        pltpu.sync_copy(day_hbm, day_v)

        def mloop(m, carry):
            for ch in range(D_MODEL // L):
                sl = pl.ds(ch * L, L)
                mv = month_v[m, sl]
                for dd in range(DAY_SIZE):
                    comb_v[m * DAY_SIZE + dd, sl] = mv + day_v[dd, sl]
            return carry

        lax.fori_loop(0, MONTH_SIZE, mloop, 0)
        pltpu.sync_copy(comb_v, comb_hbm)

    plsc.subcore_barrier()

    lanes = lax.iota(jnp.int32, L)

    def tf_copy(j, b):
        return pltpu.make_async_copy(
            tf_hbm.at[pl.ds((base + j * GROUP) * 3, GROUP * 3)], tf_v[b],
            sem_tf[b])

    def wr_copy(j, b):
        return pltpu.make_async_copy(
            rows_v[b], out_hbm.at[pl.ds(base + j * GROUP, GROUP)], sem_wr[b])

    for b in range(NBUF):
        tf_copy(b, b).start()

    def gloop(jj, carry):
        for b in range(NBUF):
            j = jj * NBUF + b
            tf_copy(j, b).wait()
            for k in range(GROUP // L):
                pos = lanes * 3 + (k * L * 3)
                m = plsc.load_gather(tf_v[b], [pos])
                d = plsc.load_gather(tf_v[b], [pos + 1])
                idx_v[b][pl.ds(k * L, L)] = m * DAY_SIZE + d

            @pl.when(jj <= NGROUPS // NBUF - 2)
            def _prefetch():
                tf_copy(j + NBUF, b).start()

            @pl.when(jj >= 1)
            def _drain_prev_write():
                wr_copy(j, b).wait()

            pltpu.async_copy(comb_hbm.at[idx_v[b]], rows_v[b], sem_g[b]).wait()
            wr_copy(j, b).start()
        return carry

    lax.fori_loop(0, NGROUPS // NBUF, gloop, 0)
    for b in range(NBUF):
        wr_copy(NGROUPS - NBUF + b, b).wait()


@functools.partial(
    pl.kernel,
    out_type=[
        jax.ShapeDtypeStruct((N_ROWS, D_MODEL), jnp.float32),
        jax.ShapeDtypeStruct((COMB, D_MODEL), jnp.float32),
    ],
    mesh=plsc.VectorSubcoreMesh(core_axis_name="c", subcore_axis_name="s"),
    compiler_params=pltpu.CompilerParams(needs_layout_passes=False),
    scratch_types=[
        pltpu.VMEM((MONTH_SIZE, D_MODEL), jnp.float32),
        pltpu.VMEM((DAY_SIZE, D_MODEL), jnp.float32),
        pltpu.VMEM((COMB, D_MODEL), jnp.float32),
        pltpu.VMEM((GROUP * 3,), jnp.int32),
        pltpu.VMEM((GROUP * 3,), jnp.int32),
        pltpu.VMEM((GROUP,), jnp.int32),
        pltpu.VMEM((GROUP,), jnp.int32),
        pltpu.VMEM((GROUP, D_MODEL), jnp.float32),
        pltpu.VMEM((GROUP, D_MODEL), jnp.float32),
        pltpu.SemaphoreType.DMA,
        pltpu.SemaphoreType.DMA,
        pltpu.SemaphoreType.DMA,
        pltpu.SemaphoreType.DMA,
        pltpu.SemaphoreType.DMA,
        pltpu.SemaphoreType.DMA,
    ],
)
def _sc_embed(tf_hbm, month_hbm, day_hbm, out_hbm, comb_hbm, *scratch):
    _sc_body(tf_hbm, month_hbm, day_hbm, out_hbm, comb_hbm, *scratch)


def kernel(time_features, month_table, day_table, weekday_table):
    tf = time_features.astype(jnp.int32).reshape(-1)
    out, _ = _sc_embed(tf, month_table, day_table)
    return out.reshape(BATCH, SEQ, D_MODEL)


# speedup vs baseline: 2.1553x; 1.4858x over previous
import functools

import jax
import jax.numpy as jnp
from jax import lax
from jax.experimental import pallas as pl
from jax.experimental.pallas import tpu as pltpu
from jax.experimental.pallas import tpu_sc as plsc

NC = 2
NS = 16
NW = NC * NS
L = 16

D_MODEL = 128
MONTH_SIZE = 13
DAY_SIZE = 32
COMB = MONTH_SIZE * DAY_SIZE

BATCH = 4096
SEQ = 200
N_ROWS = BATCH * SEQ
ROWS_PER_TILE = N_ROWS // NW
GROUP = 128
NGROUPS = ROWS_PER_TILE // GROUP
NBUF = 4
NCHUNK = 4
CHUNK_ROWS = ROWS_PER_TILE // NCHUNK


def _sc_body(tf_hbm, month_hbm, day_hbm, out_hbm,
             month_v, day_v, chunk_v, comb_sp, tf_v, idx_v,
             rows0, rows1, rows2, rows3,
             sg0, sg1, sg2, sg3, sw0, sw1, sw2, sw3):
    rows_v = (rows0, rows1, rows2, rows3)
    sem_g = (sg0, sg1, sg2, sg3)
    sem_w = (sw0, sw1, sw2, sw3)
    cid = lax.axis_index("c")
    sid = lax.axis_index("s")
    wid = sid * NC + cid
    base = wid * ROWS_PER_TILE

    @pl.when(sid == 0)
    def _build():
        pltpu.sync_copy(month_hbm, month_v)
        pltpu.sync_copy(day_hbm, day_v)

        def mloop(m, carry):
            for ch in range(D_MODEL // L):
                sl = pl.ds(ch * L, L)
                mv = month_v[m, sl]
                for dd in range(DAY_SIZE):
                    chunk_v[dd, sl] = mv + day_v[dd, sl]
            pltpu.sync_copy(chunk_v, comb_sp.at[pl.ds(m * DAY_SIZE, DAY_SIZE)])
            return carry

        lax.fori_loop(0, MONTH_SIZE, mloop, 0)

    plsc.subcore_barrier()

    lanes = lax.iota(jnp.int32, L)

    def chunk_pass(c, carry):
        pltpu.sync_copy(
            tf_hbm.at[pl.ds((base + c * CHUNK_ROWS) * 3, CHUNK_ROWS * 3)],
            tf_v)

        def dloop(k, carry2):
            pos = lanes * 3 + k * (L * 3)
            m = plsc.load_gather(tf_v, [pos])
            d = plsc.load_gather(tf_v, [pos + 1])
            idx_v[pl.ds(c * CHUNK_ROWS + k * L, L)] = m * DAY_SIZE + d
            return carry2

        return lax.fori_loop(0, CHUNK_ROWS // L, dloop, carry)

    lax.fori_loop(0, NCHUNK, chunk_pass, 0)

    def g_copy(j, b):
        return pltpu.make_async_copy(
            comb_sp.at[idx_v.at[pl.ds(j * GROUP, GROUP)]], rows_v[b],
            sem_g[b])

    def w_copy(j, b):
        return pltpu.make_async_copy(
            rows_v[b], out_hbm.at[pl.ds(base + j * GROUP, GROUP)], sem_w[b])

    for b in range(NBUF):
        g_copy(b, b).start()

    def gloop(jj, carry):
        for b in range(NBUF):
            j = jj * NBUF + b
            g_copy(j, b).wait()
            w_copy(j, b).start()
        for b in range(NBUF):
            j = jj * NBUF + b

            @pl.when(jj <= NGROUPS // NBUF - 2)
            def _refill():
                w_copy(j, b).wait()
                g_copy(j + NBUF, b).start()
        return carry

    lax.fori_loop(0, NGROUPS // NBUF, gloop, 0)
    for b in range(NBUF):
        w_copy(NGROUPS - NBUF + b, b).wait()


@functools.partial(
    pl.kernel,
    out_type=jax.ShapeDtypeStruct((N_ROWS, D_MODEL), jnp.float32),
    mesh=plsc.VectorSubcoreMesh(core_axis_name="c", subcore_axis_name="s"),
    compiler_params=pltpu.CompilerParams(needs_layout_passes=False),
    scratch_types=[
        pltpu.VMEM((MONTH_SIZE, D_MODEL), jnp.float32),
        pltpu.VMEM((DAY_SIZE, D_MODEL), jnp.float32),
        pltpu.VMEM((DAY_SIZE, D_MODEL), jnp.float32),
        pltpu.VMEM_SHARED((COMB, D_MODEL), jnp.float32),
        pltpu.VMEM((CHUNK_ROWS * 3,), jnp.int32),
        pltpu.VMEM((ROWS_PER_TILE,), jnp.int32),
        pltpu.VMEM((GROUP, D_MODEL), jnp.float32),
        pltpu.VMEM((GROUP, D_MODEL), jnp.float32),
        pltpu.VMEM((GROUP, D_MODEL), jnp.float32),
        pltpu.VMEM((GROUP, D_MODEL), jnp.float32),
        pltpu.SemaphoreType.DMA,
        pltpu.SemaphoreType.DMA,
        pltpu.SemaphoreType.DMA,
        pltpu.SemaphoreType.DMA,
        pltpu.SemaphoreType.DMA,
        pltpu.SemaphoreType.DMA,
        pltpu.SemaphoreType.DMA,
        pltpu.SemaphoreType.DMA,
    ],
)
def _sc_embed(tf_hbm, month_hbm, day_hbm, out_hbm, *scratch):
    _sc_body(tf_hbm, month_hbm, day_hbm, out_hbm, *scratch)


def kernel(time_features, month_table, day_table, weekday_table):
    tf = time_features.astype(jnp.int32).reshape(-1)
    out = _sc_embed(tf, month_table, day_table)
    return out.reshape(BATCH, SEQ, D_MODEL)


# speedup vs baseline: 2.1562x; 1.0004x over previous
import functools

import jax
import jax.numpy as jnp
from jax import lax
from jax.experimental import pallas as pl
from jax.experimental.pallas import tpu as pltpu
from jax.experimental.pallas import tpu_sc as plsc

NC = 2
NS = 16
NW = NC * NS
L = 16

D_MODEL = 128
MONTH_SIZE = 13
DAY_SIZE = 32
COMB = MONTH_SIZE * DAY_SIZE

BATCH = 4096
SEQ = 200
N_ROWS = BATCH * SEQ
ROWS_PER_TILE = N_ROWS // NW
GROUP = 128
NGROUPS = ROWS_PER_TILE // GROUP
NBUF = 4
NCHUNK = 4
CHUNK_ROWS = ROWS_PER_TILE // NCHUNK


def _sc_body(tf_hbm, month_hbm, day_hbm, out_hbm,
             month_v, day_v, chunk_v, comb_sp, tf_v, idx_v,
             rows0, rows1, rows2, rows3,
             sg0, sg1, sg2, sg3, sw0, sw1, sw2, sw3):
    rows_v = (rows0, rows1, rows2, rows3)
    sem_g = (sg0, sg1, sg2, sg3)
    sem_w = (sw0, sw1, sw2, sw3)
    cid = lax.axis_index("c")
    sid = lax.axis_index("s")
    wid = sid * NC + cid
    base = wid * ROWS_PER_TILE

    @pl.when(sid == 0)
    def _build():
        pltpu.sync_copy(month_hbm, month_v)
        pltpu.sync_copy(day_hbm, day_v)

        def mloop(m, carry):
            for ch in range(D_MODEL // L):
                sl = pl.ds(ch * L, L)
                mv = month_v[m, sl]
                for dd in range(DAY_SIZE):
                    chunk_v[dd, sl] = mv + day_v[dd, sl]
            pltpu.sync_copy(chunk_v, comb_sp.at[pl.ds(m * DAY_SIZE, DAY_SIZE)])
            return carry

        lax.fori_loop(0, MONTH_SIZE, mloop, 0)

    plsc.subcore_barrier()

    lanes = lax.iota(jnp.int32, L)

    def chunk_pass(c, carry):
        pltpu.sync_copy(
            tf_hbm.at[pl.ds((base + c * CHUNK_ROWS) * 3, CHUNK_ROWS * 3)],
            tf_v)

        def dloop(k, carry2):
            pos = lanes * 3 + k * (L * 3)
            m = plsc.load_gather(tf_v, [pos])
            d = plsc.load_gather(tf_v, [pos + 1])
            idx_v[pl.ds(c * CHUNK_ROWS + k * L, L)] = m * DAY_SIZE + d
            return carry2

        return lax.fori_loop(0, CHUNK_ROWS // L, dloop, carry)

    lax.fori_loop(0, NCHUNK, chunk_pass, 0)

    def g_copy(j, b):
        return pltpu.make_async_copy(
            comb_sp.at[idx_v.at[pl.ds(j * GROUP, GROUP)]], rows_v[b],
            sem_g[b])

    def w_copy(j, b):
        return pltpu.make_async_copy(
            rows_v[b], out_hbm.at[pl.ds(base + j * GROUP, GROUP)], sem_w[b])

    LOOKAHEAD = NBUF // 2
    for b in range(LOOKAHEAD):
        g_copy(b, b).start()

    def gloop(jj, carry):
        for b in range(NBUF):
            j = jj * NBUF + b
            g_copy(j, b).wait()
            w_copy(j, b).start()
            bn = (b + LOOKAHEAD) % NBUF

            @pl.when(j + LOOKAHEAD <= NGROUPS - 1)
            def _refill():
                @pl.when(j >= LOOKAHEAD)
                def _drain():
                    w_copy(j - LOOKAHEAD, bn).wait()

                g_copy(j + LOOKAHEAD, bn).start()
        return carry

    lax.fori_loop(0, NGROUPS // NBUF, gloop, 0)
    for b in range(NBUF):
        jt = NGROUPS - NBUF + b
        w_copy(jt, jt % NBUF).wait()


@functools.partial(
    pl.kernel,
    out_type=jax.ShapeDtypeStruct((N_ROWS, D_MODEL), jnp.float32),
    mesh=plsc.VectorSubcoreMesh(core_axis_name="c", subcore_axis_name="s"),
    compiler_params=pltpu.CompilerParams(needs_layout_passes=False),
    scratch_types=[
        pltpu.VMEM((MONTH_SIZE, D_MODEL), jnp.float32),
        pltpu.VMEM((DAY_SIZE, D_MODEL), jnp.float32),
        pltpu.VMEM((DAY_SIZE, D_MODEL), jnp.float32),
        pltpu.VMEM_SHARED((COMB, D_MODEL), jnp.float32),
        pltpu.VMEM((CHUNK_ROWS * 3,), jnp.int32),
        pltpu.VMEM((ROWS_PER_TILE,), jnp.int32),
        pltpu.VMEM((GROUP, D_MODEL), jnp.float32),
        pltpu.VMEM((GROUP, D_MODEL), jnp.float32),
        pltpu.VMEM((GROUP, D_MODEL), jnp.float32),
        pltpu.VMEM((GROUP, D_MODEL), jnp.float32),
        pltpu.SemaphoreType.DMA,
        pltpu.SemaphoreType.DMA,
        pltpu.SemaphoreType.DMA,
        pltpu.SemaphoreType.DMA,
        pltpu.SemaphoreType.DMA,
        pltpu.SemaphoreType.DMA,
        pltpu.SemaphoreType.DMA,
        pltpu.SemaphoreType.DMA,
    ],
)
def _sc_embed(tf_hbm, month_hbm, day_hbm, out_hbm, *scratch):
    _sc_body(tf_hbm, month_hbm, day_hbm, out_hbm, *scratch)


def kernel(time_features, month_table, day_table, weekday_table):
    tf = time_features.astype(jnp.int32).reshape(-1)
    out = _sc_embed(tf, month_table, day_table)
    return out.reshape(BATCH, SEQ, D_MODEL)
